# baseline (speedup 1.0000x reference)
import functools

import jax
import jax.numpy as jnp
from jax import lax
from jax.experimental import pallas as pl
from jax.experimental.pallas import tpu as pltpu
from jax.experimental.pallas import tpu_sc as plsc

NC = 2
NS = 16
LANES = 16
NBUF = 2
G = 10


def _sc_aggregate(x, src, dst, edge_attr, K=40):
    N, D = x.shape
    E = src.shape[0]
    NW = NC * NS
    e_per_tile = E // NW
    steps = e_per_tile // K
    RPT = (N // NS) // 8 * 8
    TAIL = N - NS * RPT
    zeros = jnp.zeros((N, D), jnp.float32)
    src_r = src.reshape(NW, steps // G, G, K)
    dst_r = dst.reshape(NW, steps // G, G, K)
    ea_r = edge_attr.reshape(NW, steps, K, D)
    mesh = plsc.VectorSubcoreMesh(core_axis_name="c", subcore_axis_name="s")

    @functools.partial(
        pl.kernel,
        out_type=jax.ShapeDtypeStruct((NC, N, D), jnp.float32),
        mesh=mesh,
        scratch_types=[
            pltpu.VMEM((2, G, K), jnp.int32),
            pltpu.VMEM((2, G, K), jnp.int32),
            pltpu.VMEM((3, NBUF, K, D), jnp.float32),
            pltpu.VMEM_SHARED((N, D), jnp.float32),
            pltpu.SemaphoreType.DMA((2,)),
            pltpu.SemaphoreType.DMA((2,)),
            pltpu.SemaphoreType.DMA((NBUF,)),
            pltpu.SemaphoreType.DMA((NBUF,)),
            pltpu.SemaphoreType.DMA((NBUF,)),
        ],
    )
    def body(x_hbm, src_hbm, dst_hbm, ea_hbm, zero_hbm, out_hbm,
             sidx, didx, buf, aggr_sh, s_sem, d_sem, ld_sem, g_sem, sc_sem):
        xr = buf.at[0]
        ea = buf.at[1]
        m = buf.at[2]
        c = lax.axis_index("c")
        s = lax.axis_index("s")
        tid = c * NS + s
        pltpu.sync_copy(zero_hbm.at[pl.ds(s * RPT, RPT)],
                        aggr_sh.at[pl.ds(s * RPT, RPT)])
        if TAIL:
            @pl.when(s == 0)
            def _():
                pltpu.sync_copy(zero_hbm.at[pl.ds(NS * RPT, TAIL)],
                                aggr_sh.at[pl.ds(NS * RPT, TAIL)])
        plsc.subcore_barrier()

        for k in range(2):
            pltpu.async_copy(src_hbm.at[tid, k], sidx.at[k], s_sem.at[k])
            pltpu.async_copy(dst_hbm.at[tid, k], didx.at[k], d_sem.at[k])
        for b in range(NBUF):
            pltpu.async_copy(ea_hbm.at[tid, b], ea.at[b], ld_sem.at[b])
        pltpu.make_async_copy(src_hbm.at[tid, 0], sidx.at[0], s_sem.at[0]).wait()
        for b in range(NBUF):
            pltpu.async_copy(x_hbm.at[sidx.at[0, b]], xr.at[b], g_sem.at[b])

        @pl.loop(0, steps, step=NBUF)
        def _step(i0):
            for b in range(NBUF):
                i = i0 + b
                gsub = lax.rem(i, G)
                gslot = lax.rem(lax.div(i, G), 2)
                pltpu.make_async_copy(ea_hbm.at[tid, i], ea.at[b],
                                      ld_sem.at[b]).wait()
                pltpu.make_async_copy(x_hbm.at[sidx.at[gslot, gsub]], xr.at[b],
                                      g_sem.at[b]).wait()

                @pl.when(i0 >= NBUF)
                def _():
                    pltpu.make_async_copy(m.at[b], aggr_sh.at[didx.at[gslot, gsub]],
                                          sc_sem.at[b]).wait()

                if b == 1:
                    @pl.when((gsub == 1) & (i0 + 2 * G - 1 < steps) & (i0 >= NBUF))
                    def _():
                        nslot = 1 - gslot
                        sup = lax.div(i, G) + 1
                        pltpu.async_copy(src_hbm.at[tid, sup], sidx.at[nslot],
                                         s_sem.at[nslot])
                        pltpu.async_copy(dst_hbm.at[tid, sup], didx.at[nslot],
                                         d_sem.at[nslot])

                @plsc.parallel_loop(0, K, unroll=4)
                def _row(r):
                    for j in range(D // LANES):
                        sl = pl.ds(j * LANES, LANES)
                        m[b, r, sl] = jnp.maximum(xr[b, r, sl] + ea[b, r, sl], 0.0)

                @pl.when(gsub == 0)
                def _():
                    pltpu.make_async_copy(dst_hbm.at[tid, 0], didx.at[gslot],
                                          d_sem.at[gslot]).wait()
                pltpu.async_copy(m.at[b], aggr_sh.at[didx.at[gslot, gsub]],
                                 sc_sem.at[b], add=True)

                @pl.when(i + NBUF < steps)
                def _():
                    pltpu.async_copy(ea_hbm.at[tid, i + NBUF], ea.at[b],
                                     ld_sem.at[b])
                    ip = i + NBUF
                    psub = lax.rem(ip, G)
                    pslot = lax.rem(lax.div(ip, G), 2)
                    @pl.when(psub == 0)
                    def _():
                        pltpu.make_async_copy(src_hbm.at[tid, 0], sidx.at[pslot],
                                              s_sem.at[pslot]).wait()
                    pltpu.async_copy(x_hbm.at[sidx.at[pslot, psub]], xr.at[b],
                                     g_sem.at[b])

        for b in range(NBUF):
            j = steps - NBUF + b
            pltpu.make_async_copy(m.at[b],
                                  aggr_sh.at[didx.at[(j // G) % 2, j % G]],
                                  sc_sem.at[b]).wait()

        plsc.subcore_barrier()
        pltpu.sync_copy(aggr_sh.at[pl.ds(s * RPT, RPT)],
                        out_hbm.at[c, pl.ds(s * RPT, RPT)])
        if TAIL:
            @pl.when(s == 0)
            def _():
                pltpu.sync_copy(aggr_sh.at[pl.ds(NS * RPT, TAIL)],
                                out_hbm.at[c, pl.ds(NS * RPT, TAIL)])

    return body(x, src_r, dst_r, ea_r, zeros)


def _mlp(x, p0, p1, W1, b1, W2, b2, eps, R=1000):
    N, D = x.shape
    H = W1.shape[1]
    O = W2.shape[1]

    def body(eps_ref, x_ref, p0_ref, p1_ref, w1_ref, b1_ref, w2_ref, b2_ref, out_ref):
        a = (1.0 + eps_ref[0]) * x_ref[...] + p0_ref[...] + p1_ref[...]
        h = jnp.maximum(
            jnp.dot(a, w1_ref[...], preferred_element_type=jnp.float32) + b1_ref[...], 0.0)
        out_ref[...] = jnp.maximum(
            jnp.dot(h, w2_ref[...], preferred_element_type=jnp.float32) + b2_ref[...], 0.0)

    return pl.pallas_call(
        body,
        grid=(N // R,),
        in_specs=[
            pl.BlockSpec(memory_space=pltpu.SMEM),
            pl.BlockSpec((R, D), lambda i: (i, 0)),
            pl.BlockSpec((R, D), lambda i: (i, 0)),
            pl.BlockSpec((R, D), lambda i: (i, 0)),
            pl.BlockSpec((D, H), lambda i: (0, 0)),
            pl.BlockSpec((1, H), lambda i: (0, 0)),
            pl.BlockSpec((H, O), lambda i: (0, 0)),
            pl.BlockSpec((1, O), lambda i: (0, 0)),
        ],
        out_specs=pl.BlockSpec((R, O), lambda i: (i, 0)),
        out_shape=jax.ShapeDtypeStruct((N, O), jnp.float32),
    )(eps.reshape(1), x, p0, p1, W1, b1.reshape(1, H), W2, b2.reshape(1, O))


def kernel(x, edge_index, edge_attr, W1, b1, W2, b2, eps):
    src = edge_index[0]
    dst = edge_index[1]
    partials = _sc_aggregate(x, src, dst, edge_attr)
    return _mlp(x, partials[0], partials[1], W1, b1, W2, b2, eps)
